# Initial kernel scaffold; baseline (speedup 1.0000x reference)
import jax
import jax.numpy as jnp
from jax.experimental import pallas as pl


def kernel(x, edge_index, batch, W_enc, b_enc, W1, b1, g1, be1, W2, b2, t, ln_g, ln_b, W_out, b_out):
    raise NotImplementedError("write your pallas kernel here")


# speedup vs baseline: 9.1916x; 9.1916x over previous
import functools

import jax
import jax.numpy as jnp
from jax import lax
from jax.experimental import pallas as pl
from jax.experimental.pallas import tpu as pltpu
from jax.experimental.pallas import tpu_sc as plsc

N = 10000
E = 320000
D = 128
H = 128
OUT = 128
L = 3
EPS = 1e-7

BN = 1000
EROWS = E // 128
NSUB = 16
RB = 80
NCHUNK = N // RB


def _ln(h, g, b):
    mu = jnp.mean(h, axis=-1, keepdims=True)
    var = jnp.mean((h - mu) ** 2, axis=-1, keepdims=True)
    return (h - mu) / jnp.sqrt(var + 1e-5) * g + b


def _payload(v, t):
    m = jnp.maximum(v, 0.0) + EPS
    ex = jnp.exp(m * t)
    p = m * ex
    pay0 = jnp.concatenate([p[:, :64], ex[:, :64]], axis=1)
    pay1 = jnp.concatenate([p[:, 64:], ex[:, 64:]], axis=1)
    return jnp.stack([pay0, pay1], axis=1)


def _enc_body(t_ref, x_ref, We_ref, be_ref, h_ref, p_ref):
    h = jnp.dot(x_ref[...], We_ref[...], preferred_element_type=jnp.float32)
    h = h + be_ref[...]
    h_ref[...] = h
    p_ref[...] = _payload(h, t_ref[0, 0])


def _enc_pre(x, W_enc, b_enc, t0):
    grid = (N // BN,)
    return pl.pallas_call(
        _enc_body,
        grid=grid,
        in_specs=[
            pl.BlockSpec((1, 1), lambda i: (0, 0)),
            pl.BlockSpec((BN, D), lambda i: (i, 0)),
            pl.BlockSpec((D, H), lambda i: (0, 0)),
            pl.BlockSpec((1, H), lambda i: (0, 0)),
        ],
        out_specs=[
            pl.BlockSpec((BN, H), lambda i: (i, 0)),
            pl.BlockSpec((BN, 2, H), lambda i: (i, 0, 0)),
        ],
        out_shape=[
            jax.ShapeDtypeStruct((N, H), jnp.float32),
            jax.ShapeDtypeStruct((N, 2, H), jnp.float32),
        ],
    )(t0.reshape(1, 1), x, W_enc, b_enc.reshape(1, H))


def _mid_body(t_ref, agg_ref, g_ref, hp_ref, W1_ref, b1_ref, g1_ref, be1_ref,
              W2_ref, b2_ref, lng_ref, lnb_ref, h_ref, r_ref, p_ref, *,
              has_prev):
    agg = jnp.concatenate([agg_ref[0], agg_ref[1]], axis=1)
    out0 = agg + g_ref[...]
    z = jnp.dot(out0, W1_ref[...], preferred_element_type=jnp.float32)
    z = _ln(z + b1_ref[...], g1_ref[...], be1_ref[...])
    z = jnp.maximum(z, 0.0)
    z = jnp.dot(z, W2_ref[...], preferred_element_type=jnp.float32)
    z = z + b2_ref[...]
    h_new = z + hp_ref[...] if has_prev else z
    h_ref[...] = h_new
    r = jnp.maximum(_ln(h_new, lng_ref[...], lnb_ref[...]), 0.0)
    r_ref[...] = r
    p_ref[...] = _payload(r, t_ref[0, 0])


def _mid(agg, g, hprev, W1, b1, g1, be1, W2, b2, lng, lnb, tnext, has_prev):
    grid = (N // BN,)
    full = lambda shape: pl.BlockSpec(shape, lambda i: tuple(0 for _ in shape))
    args = [tnext.reshape(1, 1), agg, g]
    in_specs = [
        pl.BlockSpec((1, 1), lambda i: (0, 0)),
        pl.BlockSpec((2, BN, 64), lambda i: (0, i, 0)),
        pl.BlockSpec((BN, H), lambda i: (i, 0)),
    ]
    if has_prev:
        args.append(hprev)
        in_specs.append(pl.BlockSpec((BN, H), lambda i: (i, 0)))
    else:
        args.append(g)
        in_specs.append(pl.BlockSpec((BN, H), lambda i: (i, 0)))
    args += [W1, b1.reshape(1, 2 * H), g1.reshape(1, 2 * H),
             be1.reshape(1, 2 * H), W2, b2.reshape(1, H),
             lng.reshape(1, H), lnb.reshape(1, H)]
    in_specs += [full((H, 2 * H)), full((1, 2 * H)), full((1, 2 * H)),
                 full((1, 2 * H)), full((2 * H, H)), full((1, H)),
                 full((1, H)), full((1, H))]
    return pl.pallas_call(
        functools.partial(_mid_body, has_prev=has_prev),
        grid=grid,
        in_specs=in_specs,
        out_specs=[
            pl.BlockSpec((BN, H), lambda i: (i, 0)),
            pl.BlockSpec((BN, H), lambda i: (i, 0)),
            pl.BlockSpec((BN, 2, H), lambda i: (i, 0, 0)),
        ],
        out_shape=[
            jax.ShapeDtypeStruct((N, H), jnp.float32),
            jax.ShapeDtypeStruct((N, H), jnp.float32),
            jax.ShapeDtypeStruct((N, 2, H), jnp.float32),
        ],
    )(*args)


def _fin_body(agg_ref, g_ref, hp_ref, W1_ref, b1_ref, g1_ref, be1_ref,
              W2_ref, b2_ref, lng_ref, lnb_ref, Wo_ref, bo_ref, y_ref):
    agg = jnp.concatenate([agg_ref[0], agg_ref[1]], axis=1)
    out0 = agg + g_ref[...]
    z = jnp.dot(out0, W1_ref[...], preferred_element_type=jnp.float32)
    z = _ln(z + b1_ref[...], g1_ref[...], be1_ref[...])
    z = jnp.maximum(z, 0.0)
    z = jnp.dot(z, W2_ref[...], preferred_element_type=jnp.float32)
    h_new = z + b2_ref[...] + hp_ref[...]
    r = jnp.maximum(_ln(h_new, lng_ref[...], lnb_ref[...]), 0.0)
    y = jnp.dot(r, Wo_ref[...], preferred_element_type=jnp.float32)
    y_ref[...] = y + bo_ref[...]


def _fin(agg, g, hprev, W1, b1, g1, be1, W2, b2, lng, lnb, W_out, b_out):
    grid = (N // BN,)
    full = lambda shape: pl.BlockSpec(shape, lambda i: tuple(0 for _ in shape))
    return pl.pallas_call(
        _fin_body,
        grid=grid,
        in_specs=[
            pl.BlockSpec((2, BN, 64), lambda i: (0, i, 0)),
            pl.BlockSpec((BN, H), lambda i: (i, 0)),
            pl.BlockSpec((BN, H), lambda i: (i, 0)),
            full((H, 2 * H)), full((1, 2 * H)), full((1, 2 * H)),
            full((1, 2 * H)), full((2 * H, H)), full((1, H)),
            full((1, H)), full((1, H)), full((H, OUT)), full((1, OUT)),
        ],
        out_specs=pl.BlockSpec((BN, OUT), lambda i: (i, 0)),
        out_shape=jax.ShapeDtypeStruct((N, OUT), jnp.float32),
    )(agg, g, hprev, W1, b1.reshape(1, 2 * H), g1.reshape(1, 2 * H),
      be1.reshape(1, 2 * H), W2, b2.reshape(1, H), lng.reshape(1, H),
      lnb.reshape(1, H), W_out, b_out.reshape(1, OUT))


def _sc_agg_kernel(P_hbm, src_hbm, dst_hbm, agg_hbm,
                   srcv, dstv, idxg, buf, outb, acc, sem):
    c = lax.axis_index("c")
    s = lax.axis_index("s")

    def zrow(r, _):
        for k in range(8):
            buf[r, pl.ds(k * 16, 16)] = jnp.zeros((16,), jnp.float32)
        return 0
    lax.fori_loop(0, RB, zrow, 0)
    for qq in range(pl.cdiv(NCHUNK, NSUB)):
        q = s + qq * NSUB

        @pl.when(q < NCHUNK)
        def _():
            pltpu.sync_copy(buf.at[pl.ds(0, RB)], acc.at[pl.ds(q * RB, RB)])
    plsc.subcore_barrier()

    lo = (s * EROWS) // NSUB
    hi = ((s + 1) * EROWS) // NSUB

    def erow(j, _):
        pltpu.sync_copy(src_hbm.at[j], srcv)
        pltpu.sync_copy(dst_hbm.at[j], dstv)
        for k in range(8):
            sl = pl.ds(k * 16, 16)
            idxg[sl] = srcv[sl] * 2 + c
        pltpu.async_copy(P_hbm.at[idxg], buf, sem).wait()
        pltpu.sync_copy(buf, acc.at[dstv], add=True)
        return 0
    lax.fori_loop(lo, hi, erow, 0)
    plsc.subcore_barrier()

    for qq in range(pl.cdiv(NCHUNK, NSUB)):
        q = s + qq * NSUB

        @pl.when(q < NCHUNK)
        def _():
            base = q * RB
            pltpu.sync_copy(acc.at[pl.ds(base, RB)], buf.at[pl.ds(0, RB)])

            def drow(r, _):
                for k in range(4):
                    num = buf[r, pl.ds(k * 16, 16)]
                    den = buf[r, pl.ds(64 + k * 16, 16)]
                    outb[r, pl.ds(k * 16, 16)] = num / (den + 1e-16)
                return 0
            lax.fori_loop(0, RB, drow, 0)
            pltpu.sync_copy(outb, agg_hbm.at[c, pl.ds(base, RB)])


@functools.cache
def _sc_agg_call():
    @functools.partial(
        pl.kernel,
        out_type=jax.ShapeDtypeStruct((2, N, 64), jnp.float32),
        mesh=plsc.VectorSubcoreMesh(core_axis_name="c", subcore_axis_name="s"),
        scratch_types=[
            pltpu.VMEM((128,), jnp.int32),
            pltpu.VMEM((128,), jnp.int32),
            pltpu.VMEM((128,), jnp.int32),
            pltpu.VMEM((128, 128), jnp.float32),
            pltpu.VMEM((RB, 64), jnp.float32),
            pltpu.VMEM_SHARED((N, 128), jnp.float32),
            pltpu.SemaphoreType.DMA,
        ],
    )
    def call(P_hbm, src_hbm, dst_hbm, agg_hbm, *scratch):
        _sc_agg_kernel(P_hbm, src_hbm, dst_hbm, agg_hbm, *scratch)
    return call


def _sc_agg(P2, src2d, dst2d):
    return _sc_agg_call()(P2, src2d, dst2d)


def kernel(x, edge_index, batch, W_enc, b_enc, W1, b1, g1, be1, W2, b2, t,
           ln_g, ln_b, W_out, b_out):
    src2d = edge_index[0].reshape(EROWS, 128)
    dst2d = edge_index[1].reshape(EROWS, 128)

    h0, P0 = _enc_pre(x, W_enc, b_enc, t[0])
    agg0 = _sc_agg(P0.reshape(2 * N, H), src2d, dst2d)
    h1, r1, P1 = _mid(agg0, h0, None, W1[0], b1[0], g1[0], be1[0], W2[0],
                      b2[0], ln_g[1], ln_b[1], t[1], has_prev=False)
    agg1 = _sc_agg(P1.reshape(2 * N, H), src2d, dst2d)
    h2, r2, P2 = _mid(agg1, r1, h1, W1[1], b1[1], g1[1], be1[1], W2[1],
                      b2[1], ln_g[2], ln_b[2], t[2], has_prev=True)
    agg2 = _sc_agg(P2.reshape(2 * N, H), src2d, dst2d)
    y = _fin(agg2, r2, h2, W1[2], b1[2], g1[2], be1[2], W2[2], b2[2],
             ln_g[0], ln_b[0], W_out, b_out)
    return y
